# baseline (speedup 1.0000x reference)
import functools

import jax
import jax.numpy as jnp
from jax import lax
from jax.experimental import pallas as pl
from jax.experimental.pallas import tpu as pltpu
from jax.experimental.pallas import tpu_sc as plsc

BATCH = 16384
EMBED = 16
HIDDEN = 128

_NC = 2
_NS = 16
_NW = _NC * _NS
_BPW = BATCH // _NW

@functools.cache
def _make_sc_gather():
    mesh = plsc.VectorSubcoreMesh(core_axis_name="c", subcore_axis_name="s")

    @functools.partial(
        pl.kernel,
        mesh=mesh,
        out_type=jax.ShapeDtypeStruct((BATCH, EMBED), jnp.float32),
        scratch_types=[
            pltpu.VMEM((_BPW,), jnp.int32),
            pltpu.VMEM((_BPW, EMBED), jnp.float32),
            pltpu.SemaphoreType.DMA,
        ],
        compiler_params=pltpu.CompilerParams(use_tc_tiling_on_sc=False),
    )
    def _sc_gather(table_hbm, idx_hbm, out_hbm, idx_v, rows_v, sem):
        wid = lax.axis_index("s") * _NC + lax.axis_index("c")
        base = wid * _BPW
        pltpu.sync_copy(idx_hbm.at[pl.ds(base, _BPW)], idx_v)
        pltpu.async_copy(table_hbm.at[idx_v], rows_v, sem).wait()
        pltpu.sync_copy(rows_v, out_hbm.at[pl.ds(base, _BPW)])

    return _sc_gather


def _dense_body(e_ref, w_ref, b_ref, g_ref, beta_ref, o_ref):
    e = e_ref[...]
    w = w_ref[...]
    y = lax.dot_general(
        e, w, (((1,), (1,)), ((), ())),
        preferred_element_type=jnp.float32,
    ) + b_ref[...]
    mean = jnp.mean(y, axis=0, keepdims=True)
    var = jnp.mean(jnp.square(y - mean), axis=0, keepdims=True)
    yn = (y - mean) * lax.rsqrt(var + 1e-5)
    yn = yn * g_ref[...] + beta_ref[...]
    sp = jnp.log1p(jnp.exp(-jnp.abs(yn))) + jnp.maximum(yn, 0.0)
    o_ref[...] = yn * jnp.tanh(sp)


@jax.jit
def _dense(e, w, b, g, beta):
    return pl.pallas_call(
        _dense_body,
        out_shape=jax.ShapeDtypeStruct((BATCH, HIDDEN), jnp.float32),
    )(e, w, b.reshape(1, HIDDEN), g.reshape(1, HIDDEN), beta.reshape(1, HIDDEN))


def kernel(x, table, W, b, gamma, beta):
    e = _make_sc_gather()(table, x.astype(jnp.int32))
    return _dense(e, W, b, gamma, beta)


# speedup vs baseline: 1.6324x; 1.6324x over previous
import functools

import jax
import jax.numpy as jnp
from jax import lax
from jax.experimental import pallas as pl
from jax.experimental.pallas import tpu as pltpu
from jax.experimental.pallas import tpu_sc as plsc

BATCH = 16384
EMBED = 16
HIDDEN = 128

_NC = 2
_NS = 16
_NW = _NC * _NS
_BPW = BATCH // _NW

@functools.cache
def _make_sc_gather():
    mesh = plsc.VectorSubcoreMesh(core_axis_name="c", subcore_axis_name="s")

    @functools.partial(
        pl.kernel,
        mesh=mesh,
        out_type=jax.ShapeDtypeStruct((BATCH, EMBED), jnp.float32),
        scratch_types=[
            pltpu.VMEM((_BPW,), jnp.int32),
            pltpu.VMEM((_BPW, EMBED), jnp.float32),
            pltpu.SemaphoreType.DMA,
        ],
    )
    def _sc_gather(table_hbm, idx_hbm, out_hbm, idx_v, rows_v, sem):
        wid = lax.axis_index("s") * _NC + lax.axis_index("c")
        base = wid * _BPW
        pltpu.sync_copy(idx_hbm.at[pl.ds(base, _BPW)], idx_v)

        def issue(g, carry):
            v = idx_v[pl.ds(g * 16, 16)]
            for k in range(16):
                r = v[k]
                pltpu.async_copy(
                    table_hbm.at[pl.ds(r, 1), :],
                    rows_v.at[pl.ds(g * 16 + k, 1), :],
                    sem,
                )
            return carry

        lax.fori_loop(0, _BPW // 16, issue, 0)
        pltpu.make_async_copy(
            out_hbm.at[pl.ds(base, _BPW)], rows_v, sem
        ).wait()
        pltpu.sync_copy(rows_v, out_hbm.at[pl.ds(base, _BPW)])

    return _sc_gather


def _dense_body(e_ref, w_ref, b_ref, g_ref, beta_ref, o_ref):
    e = e_ref[...]
    w = w_ref[...]
    y = lax.dot_general(
        e, w, (((1,), (1,)), ((), ())),
        preferred_element_type=jnp.float32,
    ) + b_ref[...]
    mean = jnp.mean(y, axis=0, keepdims=True)
    var = jnp.mean(jnp.square(y - mean), axis=0, keepdims=True)
    yn = (y - mean) * lax.rsqrt(var + 1e-5)
    yn = yn * g_ref[...] + beta_ref[...]
    sp = jnp.log1p(jnp.exp(-jnp.abs(yn))) + jnp.maximum(yn, 0.0)
    o_ref[...] = yn * jnp.tanh(sp)


@jax.jit
def _dense(e, w, b, g, beta):
    return pl.pallas_call(
        _dense_body,
        out_shape=jax.ShapeDtypeStruct((BATCH, HIDDEN), jnp.float32),
    )(e, w, b.reshape(1, HIDDEN), g.reshape(1, HIDDEN), beta.reshape(1, HIDDEN))


def kernel(x, table, W, b, gamma, beta):
    e = _make_sc_gather()(table, x.astype(jnp.int32))
    return _dense(e, W, b, gamma, beta)
